# Initial kernel scaffold; baseline (speedup 1.0000x reference)
import jax
import jax.numpy as jnp
from jax.experimental import pallas as pl


def kernel(batch_feat, batch_mask_lens, W, b):
    raise NotImplementedError("write your pallas kernel here")


# speedup vs baseline: 6.7793x; 6.7793x over previous
import functools
import math

import jax
import jax.numpy as jnp
from jax import lax
from jax.experimental import pallas as pl
from jax.experimental.pallas import tpu as pltpu

_B = 8
_N = 16
_T = 64
_C = 512
_EMB = 256
_L = _N * _T


def _main_body(lens_ref, feat_ref, w_ref, bias_ref,
               proj_ref, pos_ref, mask_ref, att_ref, cap_ref, acc_ref):
    b_idx = pl.program_id(0)

    acc_ref[...] = lax.dot_general(
        feat_ref[...], w_ref[...],
        dimension_numbers=(((1,), (1,)), ((), ())),
        preferred_element_type=jnp.float32) + bias_ref[...]

    t_f = lax.broadcasted_iota(jnp.int32, (_T, _EMB), 0).astype(jnp.float32)
    d_i = lax.broadcasted_iota(jnp.int32, (_T, _EMB), 1)
    inv_freq = jnp.exp((d_i // 4).astype(jnp.float32) *
                       (-math.log(10000.0) / (_EMB // 4)))
    ang = t_f * inv_freq
    table = jnp.where((d_i % 2) == 0, jnp.sin(ang), jnp.cos(ang))

    starts = []
    s = 0
    for n in range(_N):
        starts.append(s)
        s = s + lens_ref[b_idx, n]
    total = s

    li_col = lax.broadcasted_iota(jnp.int32, (_L, 1), 0)
    li_row = lax.broadcasted_iota(jnp.int32, (1, _L), 1)
    cnt_c = jnp.zeros((_L, 1), jnp.int32)
    cnt_r = jnp.zeros((1, _L), jnp.int32)
    spos_c = jnp.zeros((_L, 1), jnp.int32)
    for n in range(_N):
        sel_c = li_col >= starts[n]
        cnt_c += sel_c.astype(jnp.int32)
        cnt_r += (li_row >= starts[n]).astype(jnp.int32)
        if n:
            spos_c = jnp.where(sel_c, starts[n], spos_c)
    valid_c = li_col < total
    valid_r = li_row < total
    cap_c = cnt_c - 1
    cap_r = cnt_r - 1

    src_c = jnp.where(valid_c, cap_c * _T + (li_col - spos_c), -1)
    perm = (li_row == src_c).astype(jnp.float32)
    proj_ref[...] = lax.dot_general(
        perm, acc_ref[...],
        dimension_numbers=(((1,), (0,)), ((), ())),
        preferred_element_type=jnp.float32)

    pos_id_c = jnp.where(valid_c, li_col - spos_c, 0)
    t_row = lax.broadcasted_iota(jnp.int32, (1, _T), 1)
    perm_t = (t_row == pos_id_c).astype(jnp.float32)
    pos_ref[...] = lax.dot_general(
        perm_t, table,
        dimension_numbers=(((1,), (0,)), ((), ())),
        preferred_element_type=jnp.float32)

    cap_ref[...] = jnp.where(valid_r, cap_r, -1)
    mask_ref[...] = jnp.logical_not(valid_r)
    cpr = jnp.where(valid_r, cap_r, _N + li_row)
    cpc = jnp.where(valid_c, cap_c, _N + li_col)
    att_ref[...] = cpc != cpr


@jax.jit
def _run(feat_flat, lens, w, bias2d):
    out = pl.pallas_call(
        _main_body,
        grid=(_B,),
        in_specs=[
            pl.BlockSpec(memory_space=pltpu.SMEM),
            pl.BlockSpec((None, _L, _C), lambda b: (b, 0, 0)),
            pl.BlockSpec((_EMB, _C), lambda b: (0, 0)),
            pl.BlockSpec((1, _EMB), lambda b: (0, 0)),
        ],
        out_specs=[
            pl.BlockSpec((None, _L, _EMB), lambda b: (b, 0, 0)),
            pl.BlockSpec((None, _L, _EMB), lambda b: (b, 0, 0)),
            pl.BlockSpec((None, 1, _L), lambda b: (b, 0, 0)),
            pl.BlockSpec((None, _L, _L), lambda b: (b, 0, 0)),
            pl.BlockSpec((None, 1, _L), lambda b: (b, 0, 0)),
        ],
        out_shape=[
            jax.ShapeDtypeStruct((_B, _L, _EMB), jnp.float32),
            jax.ShapeDtypeStruct((_B, _L, _EMB), jnp.float32),
            jax.ShapeDtypeStruct((_B, 1, _L), jnp.bool_),
            jax.ShapeDtypeStruct((_B, _L, _L), jnp.bool_),
            jax.ShapeDtypeStruct((_B, 1, _L), jnp.int32),
        ],
        scratch_shapes=[pltpu.VMEM((_L, _EMB), jnp.float32)],
    )(lens, feat_flat, w, bias2d)
    return out


def kernel(batch_feat, batch_mask_lens, W, b):
    feat_flat = batch_feat.reshape(_B, _L, _C)
    bias2d = b.reshape(1, _EMB)
    proj, pos, mask3, att, cap3 = _run(feat_flat, batch_mask_lens, W, bias2d)
    return (proj, pos, mask3.reshape(_B, _L), att, cap3.reshape(_B, _L))
